# baseline (speedup 1.0000x reference)
import jax
import jax.numpy as jnp
from jax import lax
from jax.experimental import pallas as pl
from jax.experimental.pallas import tpu as pltpu
from jax.experimental.pallas import tpu_sc as plsc
import numpy as np

NUM_PATH = 4
OUT_DIM = 128
NUM_ELEMENTS = 8
N_NODES = 100000
ALPHA = 1.0 / np.sqrt(float(NUM_PATH))

L = 16
NC, NS = 2, 16
NW = NC * NS
C = 64
PER_W = N_NODES // NW
ITERS = (PER_W + C - 1) // C
ITERS2 = ITERS + (ITERS % 2)
ASLAB = ITERS * C
WSZ = NUM_ELEMENTS * OUT_DIM * NUM_PATH
XCH = C * OUT_DIM
XB = NUM_PATH * XCH
OCH = C * OUT_DIM


def _body(x_hbm, attrs_hbm, w_hbm, out_hbm,
          x_buf, out_buf, a_buf, w_buf, ei_vmem,
          isem0, isem1, osem0, osem1, asem):
    wid = lax.axis_index("c") * NS + lax.axis_index("s")
    start = (wid * PER_W) // 8 * 8
    end = jnp.where(wid == NW - 1, N_NODES, ((wid + 1) * PER_W) // 8 * 8)
    last_base = end - C
    astart = jnp.minimum(start, N_NODES - ASLAB)

    in_sems = (isem0, isem1)
    out_sems = (osem0, osem1)

    acps = [pltpu.make_async_copy(
                attrs_hbm.at[pl.ds(e * N_NODES + astart, ASLAB)],
                a_buf.at[pl.ds(e * ASLAB, ASLAB)], asem)
            for e in range(NUM_ELEMENTS)]
    for cp in acps:
        cp.start()
    pltpu.sync_copy(w_hbm, w_buf)
    for cp in acps:
        cp.wait()

    @plsc.parallel_loop(0, ASLAB // L, unroll=4)
    def grp_body(g):
        sl = pl.ds(g * L, L)
        best = a_buf[pl.ds(g * L, L)]
        ei = jnp.zeros((L,), jnp.int32)
        for e in range(1, NUM_ELEMENTS):
            ae = a_buf[pl.ds(e * ASLAB + g * L, L)]
            gt = ae > best
            best = jnp.where(gt, ae, best)
            ei = jnp.where(gt, jnp.full((L,), e, jnp.int32), ei)
        ei_vmem[sl] = ei

    def chunk_base(i):
        return jnp.minimum(start + i * C, last_base)

    def in_copies(i, b):
        base = chunk_base(i)
        sem = in_sems[b]
        return [pltpu.make_async_copy(
                    x_hbm.at[pl.ds(p * N_NODES * OUT_DIM + base * OUT_DIM,
                                   XCH)],
                    x_buf.at[pl.ds(b * XB + p * XCH, XCH)], sem)
                for p in range(NUM_PATH)]

    def issue_in(i, b):
        for cp in in_copies(i, b):
            cp.start()

    def wait_in(i, b):
        for cp in in_copies(i, b):
            cp.wait()

    def out_copy(i, b):
        base = chunk_base(i)
        return pltpu.make_async_copy(
            out_buf.at[pl.ds(b * OCH, OCH)],
            out_hbm.at[pl.ds(base * OUT_DIM, OCH)], out_sems[b])

    def compute(i, b):
        xo = b * XB
        oo = b * OCH
        lbase = chunk_base(i) - astart

        @plsc.parallel_loop(0, C, unroll=4)
        def node_body(n):
            se = ei_vmem[pl.ds(lbase + n, L)][0]
            wb = se * (OUT_DIM * NUM_PATH)
            for k in range(OUT_DIM // L):
                acc = None
                for p in range(NUM_PATH):
                    xv = x_buf[pl.ds(xo + p * XCH + n * OUT_DIM + k * L, L)]
                    wv = w_buf[pl.ds(wb + p * OUT_DIM + k * L, L)]
                    t = xv * wv
                    acc = t if acc is None else acc + t
                out_buf[pl.ds(oo + n * OUT_DIM + k * L, L)] = acc * ALPHA

    issue_in(0, 0)

    def outer_body(io, carry):
        for b in range(2):
            i = 2 * io + b
            wait_in(i, b)
            issue_in(i + 1, 1 - b)

            @pl.when(i >= 2)
            def _():
                out_copy(i - 2, b).wait()

            compute(i, b)
            out_copy(i, b).start()
        return carry

    lax.fori_loop(0, ITERS2 // 2, outer_body, 0)

    wait_in(ITERS2, 0)
    out_copy(ITERS2 - 2, 0).wait()
    out_copy(ITERS2 - 1, 1).wait()


def kernel(x, node_attrs, weights):
    mesh = plsc.VectorSubcoreMesh(core_axis_name="c", subcore_axis_name="s",
                                  num_cores=NC, num_subcores=NS)
    f = pl.kernel(
        _body,
        out_type=jax.ShapeDtypeStruct((N_NODES * OUT_DIM,), jnp.float32),
        mesh=mesh,
        compiler_params=pltpu.CompilerParams(needs_layout_passes=False),
        scratch_types=[
            pltpu.VMEM((2 * XB,), jnp.float32),
            pltpu.VMEM((2 * OCH,), jnp.float32),
            pltpu.VMEM((NUM_ELEMENTS * ASLAB,), jnp.float32),
            pltpu.VMEM((WSZ,), jnp.float32),
            pltpu.VMEM((ASLAB + L,), jnp.int32),
            pltpu.SemaphoreType.DMA,
            pltpu.SemaphoreType.DMA,
            pltpu.SemaphoreType.DMA,
            pltpu.SemaphoreType.DMA,
            pltpu.SemaphoreType.DMA,
        ],
    )
    w_pm = jnp.transpose(weights, (0, 2, 1))
    out_flat = f(x.reshape(-1), jnp.transpose(node_attrs).reshape(-1),
                 w_pm.reshape(-1))
    return out_flat.reshape(N_NODES, OUT_DIM)
